# baseline (speedup 1.0000x reference)
import functools

import jax
import jax.numpy as jnp
from jax.experimental import pallas as pl
from jax.experimental.pallas import tpu as pltpu

_NOB = 3000


def _body(obj_ref, dist_ref, idx_ref, out_ref, *, scale):
    i = pl.program_id(0)

    @pl.when(i == 0)
    def _():
        out_ref[0, 0] = 0.0

    s = obj_ref[0] + obj_ref[1] + obj_ref[2]
    idxf = idx_ref[...].astype(jnp.float32)
    t = idxf * (3.0 * idxf - s)
    contrib = jnp.where(t > 0.0, dist_ref[...], 0.0)
    out_ref[0, 0] += jnp.sum(contrib) * scale


def kernel(hand_xyz, hand_face, obj_xyz, nn_dist, nn_idx):
    del hand_face
    bsz = hand_xyz.shape[0]
    no = obj_xyz.shape[1]

    obj_t = jnp.transpose(obj_xyz, (2, 1, 0))
    dist_t = nn_dist.T
    idx_t = nn_idx.T

    out = pl.pallas_call(
        functools.partial(_body, scale=100.0 / bsz),
        grid=(no // _NOB,),
        in_specs=[
            pl.BlockSpec((3, _NOB, bsz), lambda i: (0, i, 0)),
            pl.BlockSpec((_NOB, bsz), lambda i: (i, 0)),
            pl.BlockSpec((_NOB, bsz), lambda i: (i, 0)),
        ],
        out_specs=pl.BlockSpec(
            (1, 1), lambda i: (0, 0), memory_space=pltpu.SMEM
        ),
        out_shape=jax.ShapeDtypeStruct((1, 1), jnp.float32),
        compiler_params=pltpu.CompilerParams(
            vmem_limit_bytes=120 * 1024 * 1024),
    )(obj_t, dist_t, idx_t)
    return out[0, 0]


# speedup vs baseline: 1.1372x; 1.1372x over previous
import functools

import jax
import jax.numpy as jnp
from jax.experimental import pallas as pl
from jax.experimental.pallas import tpu as pltpu

_NOB = 600


def _body(x_ref, y_ref, z_ref, dist_ref, idx_ref, out_ref, *, scale):
    i = pl.program_id(0)

    @pl.when(i == 0)
    def _():
        out_ref[0, 0] = 0.0

    s = x_ref[0] + y_ref[0] + z_ref[0]
    idxf = idx_ref[...].astype(jnp.float32)
    t = idxf * (3.0 * idxf - s)
    contrib = jnp.where(t > 0.0, dist_ref[...], 0.0)
    out_ref[0, 0] += jnp.sum(contrib) * scale


def kernel(hand_xyz, hand_face, obj_xyz, nn_dist, nn_idx):
    del hand_face
    bsz = hand_xyz.shape[0]
    no = obj_xyz.shape[1]

    obj_t = jnp.transpose(obj_xyz, (2, 1, 0))
    dist_t = nn_dist.T
    idx_t = nn_idx.T

    out = pl.pallas_call(
        functools.partial(_body, scale=100.0 / bsz),
        grid=(no // _NOB,),
        in_specs=[
            pl.BlockSpec((1, _NOB, bsz), lambda i: (0, i, 0)),
            pl.BlockSpec((1, _NOB, bsz), lambda i: (1, i, 0)),
            pl.BlockSpec((1, _NOB, bsz), lambda i: (2, i, 0)),
            pl.BlockSpec((_NOB, bsz), lambda i: (i, 0)),
            pl.BlockSpec((_NOB, bsz), lambda i: (i, 0)),
        ],
        out_specs=pl.BlockSpec(
            (1, 1), lambda i: (0, 0), memory_space=pltpu.SMEM
        ),
        out_shape=jax.ShapeDtypeStruct((1, 1), jnp.float32),
    )(obj_t, obj_t, obj_t, dist_t, idx_t)
    return out[0, 0]


# speedup vs baseline: 1.1446x; 1.0066x over previous
import functools

import jax
import jax.numpy as jnp
from jax.experimental import pallas as pl
from jax.experimental.pallas import tpu as pltpu

_NOB = 600


def _body(obj_ref, dist_ref, idx_ref, out_ref, *, scale):
    i = pl.program_id(0)

    @pl.when(i == 0)
    def _():
        out_ref[0, 0] = 0.0

    s = obj_ref[0] + obj_ref[1] + obj_ref[2]
    idxf = idx_ref[...].astype(jnp.float32)
    t = idxf * (3.0 * idxf - s)
    contrib = jnp.where(t > 0.0, dist_ref[...], 0.0)
    out_ref[0, 0] += jnp.sum(contrib) * scale


def kernel(hand_xyz, hand_face, obj_xyz, nn_dist, nn_idx):
    del hand_face
    bsz = hand_xyz.shape[0]
    no = obj_xyz.shape[1]

    obj_t = jnp.transpose(obj_xyz, (2, 1, 0))
    dist_t = nn_dist.T
    idx_t = nn_idx.T

    out = pl.pallas_call(
        functools.partial(_body, scale=100.0 / bsz),
        grid=(no // _NOB,),
        in_specs=[
            pl.BlockSpec((3, _NOB, bsz), lambda i: (0, i, 0)),
            pl.BlockSpec((_NOB, bsz), lambda i: (i, 0)),
            pl.BlockSpec((_NOB, bsz), lambda i: (i, 0)),
        ],
        out_specs=pl.BlockSpec(
            (1, 1), lambda i: (0, 0), memory_space=pltpu.SMEM
        ),
        out_shape=jax.ShapeDtypeStruct((1, 1), jnp.float32),
    )(obj_t, dist_t, idx_t)
    return out[0, 0]
